# Initial kernel scaffold; baseline (speedup 1.0000x reference)
import jax
import jax.numpy as jnp
from jax.experimental import pallas as pl


def kernel(data, log_weights, segment_ids):
    raise NotImplementedError("write your pallas kernel here")


# speedup vs baseline: 10.5630x; 10.5630x over previous
import functools

import jax
import jax.numpy as jnp
from jax import lax
from jax.experimental import pallas as pl
from jax.experimental.pallas import tpu as pltpu
from jax.experimental.pallas import tpu_sc as plsc

N_NODES = 10000
N_EDGES = 320000
BATCH = 128

NC, NS, LANES = 2, 16, 16
NW = NC * NS
EW = N_EDGES // NW
K = 80
CH = EW // K
N_PAD = 10112
RPS = N_PAD // NS

_MESH = plsc.VectorSubcoreMesh(core_axis_name="c", subcore_axis_name="s",
                               num_cores=NC, num_subcores=NS)
_PARAMS = pltpu.CompilerParams(needs_layout_passes=False)


def _sc_main_body(data_h, lw_h, ids3_h, acc_out_h,
                  data_v, idx_v, lw_c0, lw_c1, acc_sh, sem0, sem1):
    c = lax.axis_index("c")
    s = lax.axis_index("s")
    w = c * NS + s
    base = w * EW

    def zrow(i, carry):
        for h in range(BATCH // LANES):
            data_v[i, pl.ds(h * LANES, LANES)] = jnp.zeros((LANES,), jnp.float32)
        return carry

    lax.fori_loop(0, 2 * K, zrow, 0, unroll=False)

    arow = s * RPS
    for i in range(3):
        pltpu.sync_copy(data_v, acc_sh.at[pl.ds(arow + i * 2 * K, 2 * K)])
    pltpu.sync_copy(data_v.at[pl.ds(0, RPS - 6 * K)],
                    acc_sh.at[pl.ds(arow + 6 * K, RPS - 6 * K)])
    plsc.subcore_barrier()

    pltpu.sync_copy(ids3_h.at[w], idx_v)

    sems = (sem0, sem1)
    lwbufs = (lw_c0, lw_c1)

    def gather_descs(chunk, b):
        off = base + chunk * K
        return (
            pltpu.make_async_copy(data_h.at[pl.ds(off, K)],
                                  data_v.at[pl.ds(K * b, K)], sems[b]),
            pltpu.make_async_copy(lw_h.at[pl.ds(off, K)], lwbufs[b], sems[b]),
        )

    def gather_start(chunk, b):
        for d in gather_descs(chunk, b):
            d.start()

    def gather_wait(chunk, b):
        for d in gather_descs(chunk, b):
            d.wait()

    def compute_and_scatter(chunk, b):
        lwbuf = lwbufs[b]

        def ebody(k, carry):
            kv = jnp.broadcast_to(k, (LANES,)).astype(jnp.int32)
            lwb = plsc.load_gather(lwbuf, [kv])
            row = K * b + k
            for h in range(BATCH // LANES):
                x = data_v[row, pl.ds(h * LANES, LANES)]
                data_v[row, pl.ds(h * LANES, LANES)] = jnp.exp(x + lwb)
            return carry

        lax.fori_loop(0, K, ebody, 0, unroll=False)
        pltpu.sync_copy(data_v.at[pl.ds(K * b, K)],
                        acc_sh.at[idx_v.at[chunk]], add=True)

    gather_start(0, 0)

    def gloop(g, carry):
        for b in range(2):
            chunk = 2 * g + b
            gather_wait(chunk, b)
            gather_start(chunk + 1, 1 - b)
            compute_and_scatter(chunk, b)
        return carry

    lax.fori_loop(0, (CH - 1) // 2, gloop, 0, unroll=False)
    gather_wait(CH - 1, 0)
    compute_and_scatter(CH - 1, 0)

    plsc.subcore_barrier()
    out_base = c * N_PAD + arow
    for i in range(3):
        pltpu.sync_copy(acc_sh.at[pl.ds(arow + i * 2 * K, 2 * K)], data_v)
        pltpu.sync_copy(data_v, acc_out_h.at[pl.ds(out_base + i * 2 * K, 2 * K)])
    pltpu.sync_copy(acc_sh.at[pl.ds(arow + 6 * K, RPS - 6 * K)],
                    data_v.at[pl.ds(0, RPS - 6 * K)])
    pltpu.sync_copy(data_v.at[pl.ds(0, RPS - 6 * K)],
                    acc_out_h.at[pl.ds(out_base + 6 * K, RPS - 6 * K)])


_sc_main_call = functools.partial(
    pl.kernel,
    out_type=jax.ShapeDtypeStruct((NC * N_PAD, BATCH), jnp.float32),
    mesh=_MESH,
    compiler_params=_PARAMS,
    scratch_types=[
        pltpu.VMEM((2 * K, BATCH), jnp.float32),
        pltpu.VMEM((CH, K), jnp.int32),
        pltpu.VMEM((K,), jnp.float32),
        pltpu.VMEM((K,), jnp.float32),
        pltpu.VMEM_SHARED((N_PAD, BATCH), jnp.float32),
        pltpu.SemaphoreType.DMA,
        pltpu.SemaphoreType.DMA,
    ],
)(_sc_main_body)


def _sc_norm_body(lw_h, ids3_h, accw_out_h, lw_v, w_v, idx_v, accw_sh):
    c = lax.axis_index("c")
    s = lax.axis_index("s")
    w = c * NS + s
    base = w * EW

    def zrow_w(i, carry):
        for h in range(BATCH // LANES):
            w_v[i, pl.ds(h * LANES, LANES)] = jnp.zeros((LANES,), jnp.float32)
        return carry

    lax.fori_loop(0, K, zrow_w, 0, unroll=False)

    arow = s * RPS
    for i in range(7):
        pltpu.sync_copy(w_v, accw_sh.at[pl.ds(arow + i * K, K)])
    pltpu.sync_copy(w_v.at[pl.ds(0, RPS - 7 * K)],
                    accw_sh.at[pl.ds(arow + 7 * K, RPS - 7 * K)])
    plsc.subcore_barrier()

    pltpu.sync_copy(ids3_h.at[w], idx_v)
    pltpu.sync_copy(lw_h.at[pl.ds(base, EW)], lw_v)

    def cbody(chunk, carry):
        cbase = jnp.int32(chunk * K)

        def ebody(k, carry2):
            kv = jnp.broadcast_to(cbase + k, (LANES,)).astype(jnp.int32)
            ew = jnp.exp(plsc.load_gather(lw_v, [kv]))
            for h in range(BATCH // LANES):
                w_v[k, pl.ds(h * LANES, LANES)] = ew
            return carry2

        lax.fori_loop(0, K, ebody, 0, unroll=False)
        pltpu.sync_copy(w_v, accw_sh.at[idx_v.at[chunk]], add=True)
        return carry

    lax.fori_loop(0, CH, cbody, 0, unroll=False)

    plsc.subcore_barrier()
    out_base = c * N_PAD + arow
    for i in range(7):
        pltpu.sync_copy(accw_sh.at[pl.ds(arow + i * K, K)], w_v)
        pltpu.sync_copy(w_v, accw_out_h.at[pl.ds(out_base + i * K, K)])
    pltpu.sync_copy(accw_sh.at[pl.ds(arow + 7 * K, RPS - 7 * K)],
                    w_v.at[pl.ds(0, RPS - 7 * K)])
    pltpu.sync_copy(w_v.at[pl.ds(0, RPS - 7 * K)],
                    accw_out_h.at[pl.ds(out_base + 7 * K, RPS - 7 * K)])


_sc_norm_call = functools.partial(
    pl.kernel,
    out_type=jax.ShapeDtypeStruct((NC * N_PAD, BATCH), jnp.float32),
    mesh=_MESH,
    compiler_params=_PARAMS,
    scratch_types=[
        pltpu.VMEM((EW,), jnp.float32),
        pltpu.VMEM((K, BATCH), jnp.float32),
        pltpu.VMEM((CH, K), jnp.int32),
        pltpu.VMEM_SHARED((N_PAD, BATCH), jnp.float32),
    ],
)(_sc_norm_body)


ROWS_BLK = 1000


def _finish_body(acc_ref, accw_ref, out_ref):
    a = acc_ref[0] + acc_ref[1]
    wsum = accw_ref[0] + accw_ref[1]
    out_ref[...] = jnp.log(a) - jnp.log(wsum)


_finish_call = pl.pallas_call(
    _finish_body,
    grid=(N_NODES // ROWS_BLK,),
    in_specs=[
        pl.BlockSpec((NC, ROWS_BLK, BATCH), lambda i: (0, i, 0)),
        pl.BlockSpec((NC, ROWS_BLK, BATCH), lambda i: (0, i, 0)),
    ],
    out_specs=pl.BlockSpec((ROWS_BLK, BATCH), lambda i: (i, 0)),
    out_shape=jax.ShapeDtypeStruct((N_NODES, BATCH), jnp.float32),
)


def kernel(data, log_weights, segment_ids):
    ids3 = segment_ids.astype(jnp.int32).reshape(NW, CH, K)
    lw = log_weights.astype(jnp.float32)
    acc = _sc_main_call(data, lw, ids3)
    accw = _sc_norm_call(lw, ids3)
    acc = acc.reshape(NC, N_PAD, BATCH)[:, :N_NODES]
    accw = accw.reshape(NC, N_PAD, BATCH)[:, :N_NODES]
    return _finish_call(acc, accw)


# speedup vs baseline: 16.7733x; 1.5879x over previous
import functools

import jax
import jax.numpy as jnp
from jax import lax
from jax.experimental import pallas as pl
from jax.experimental.pallas import tpu as pltpu
from jax.experimental.pallas import tpu_sc as plsc

N_NODES = 10000
N_EDGES = 320000
BATCH = 128

NC, NS, LANES = 2, 16, 16
NW = NC * NS
EW = N_EDGES // NW
K = 40
CH = EW // K
NBUF = 3
N_PAD = 10240
RPS = N_PAD // NS
NF = N_PAD // BATCH

_MESH = plsc.VectorSubcoreMesh(core_axis_name="c", subcore_axis_name="s",
                               num_cores=NC, num_subcores=NS)
_PARAMS = pltpu.CompilerParams(needs_layout_passes=False)


def _sc_main_body(data_h, lw_h, ids3_h, acc_out_h,
                  data_v, idx_v, lw_c0, lw_c1, lw_c2, acc_sh,
                  sem0, sem1, sem2):
    c = lax.axis_index("c")
    s = lax.axis_index("s")
    w = c * NS + s
    base = w * EW

    def zrow(i, carry):
        for h in range(BATCH // LANES):
            data_v[i, pl.ds(h * LANES, LANES)] = jnp.zeros((LANES,), jnp.float32)
        return carry

    lax.fori_loop(0, NBUF * K, zrow, 0, unroll=False)

    arow = s * RPS
    for i in range(RPS // (2 * K)):
        pltpu.sync_copy(data_v.at[pl.ds(0, 2 * K)],
                        acc_sh.at[pl.ds(arow + i * 2 * K, 2 * K)])
    plsc.subcore_barrier()

    pltpu.sync_copy(ids3_h.at[w], idx_v)

    sems = (sem0, sem1, sem2)
    lwbufs = (lw_c0, lw_c1, lw_c2)

    def gather_descs(chunk, b):
        off = base + chunk * K
        return (
            pltpu.make_async_copy(data_h.at[pl.ds(off, K)],
                                  data_v.at[pl.ds(K * b, K)], sems[b]),
            pltpu.make_async_copy(lw_h.at[pl.ds(off, K)], lwbufs[b], sems[b]),
        )

    def gather_start(chunk, b):
        for d in gather_descs(chunk, b):
            d.start()

    def gather_wait(chunk, b):
        for d in gather_descs(chunk, b):
            d.wait()

    def compute(chunk, b):
        lwbuf = lwbufs[b]

        def ebody(k, carry):
            kv = jnp.broadcast_to(k, (LANES,)).astype(jnp.int32)
            lwb = plsc.load_gather(lwbuf, [kv])
            row = K * b + k
            for h in range(BATCH // LANES):
                x = data_v[row, pl.ds(h * LANES, LANES)]
                data_v[row, pl.ds(h * LANES, LANES)] = jnp.exp(x + lwb)
            return carry

        lax.fori_loop(0, K, ebody, 0, unroll=False)

    def compute_and_scatter(chunk, b):
        compute(chunk, b)
        pltpu.sync_copy(data_v.at[pl.ds(K * b, K)],
                        acc_sh.at[idx_v.at[chunk]], add=True)

    for c in range(NBUF - 1):
        gather_start(c, c)

    GBODY = CH // NBUF - 1

    def gloop(g, carry):
        for j in range(NBUF):
            chunk = g * NBUF + j
            gather_wait(chunk, j)
            gather_start(chunk + NBUF - 1, (j + NBUF - 1) % NBUF)
            compute_and_scatter(chunk, j)
        return carry

    lax.fori_loop(0, GBODY, gloop, 0, unroll=False)
    for c in range(GBODY * NBUF, CH):
        j = c % NBUF
        gather_wait(c, j)
        if c + NBUF - 1 < CH:
            gather_start(c + NBUF - 1, (j + NBUF - 1) % NBUF)
        compute_and_scatter(c, j)

    plsc.subcore_barrier()
    out_base = c * N_PAD + arow
    for i in range(RPS // (2 * K)):
        pltpu.sync_copy(acc_sh.at[pl.ds(arow + i * 2 * K, 2 * K)],
                        data_v.at[pl.ds(0, 2 * K)])
        pltpu.sync_copy(data_v.at[pl.ds(0, 2 * K)],
                        acc_out_h.at[pl.ds(out_base + i * 2 * K, 2 * K)])


_sc_main_call = functools.partial(
    pl.kernel,
    out_type=jax.ShapeDtypeStruct((NC * N_PAD, BATCH), jnp.float32),
    mesh=_MESH,
    compiler_params=_PARAMS,
    scratch_types=[
        pltpu.VMEM((NBUF * K, BATCH), jnp.float32),
        pltpu.VMEM((CH, K), jnp.int32),
        pltpu.VMEM((K,), jnp.float32),
        pltpu.VMEM((K,), jnp.float32),
        pltpu.VMEM((K,), jnp.float32),
        pltpu.VMEM_SHARED((N_PAD, BATCH), jnp.float32),
        pltpu.SemaphoreType.DMA,
        pltpu.SemaphoreType.DMA,
        pltpu.SemaphoreType.DMA,
    ],
)(_sc_main_body)


def _sc_norm_body(lw_h, ids_h, accw_out_h, lw_v, ids_v, norm2d, rowidx_v,
                  accw_sh):
    c = lax.axis_index("c")
    s = lax.axis_index("s")
    w = c * NS + s
    base = w * EW

    def zrow(i, carry):
        for h in range(BATCH // LANES):
            norm2d[i, pl.ds(h * LANES, LANES)] = jnp.zeros((LANES,), jnp.float32)
        return carry

    lax.fori_loop(0, NF, zrow, 0, unroll=False)

    @pl.when(s == 0)
    def _():
        pltpu.sync_copy(norm2d, accw_sh)

    for i in range(NF // LANES):
        rowidx_v[pl.ds(i * LANES, LANES)] = (
            lax.iota(jnp.int32, LANES) + jnp.int32(i * LANES))
    plsc.subcore_barrier()

    pltpu.sync_copy(lw_h.at[pl.ds(base, EW)], lw_v)
    pltpu.sync_copy(ids_h.at[pl.ds(base, EW)], ids_v)

    def ebody(g, carry):
        idv = ids_v[pl.ds(g * LANES, LANES)]
        v = jnp.exp(lw_v[pl.ds(g * LANES, LANES)])
        plsc.addupdate_scatter(
            norm2d, [lax.shift_right_logical(idv, 7), idv & 127], v)
        return carry

    lax.fori_loop(0, EW // LANES, ebody, 0, unroll=False)

    pltpu.sync_copy(norm2d, accw_sh.at[rowidx_v], add=True)
    plsc.subcore_barrier()

    @pl.when(s < NF // 8)
    def _():
        pltpu.sync_copy(accw_sh.at[pl.ds(s * 8, 8)],
                        accw_out_h.at[pl.ds(c * NF + s * 8, 8)])


_sc_norm_call = functools.partial(
    pl.kernel,
    out_type=jax.ShapeDtypeStruct((NC * NF, BATCH), jnp.float32),
    mesh=_MESH,
    compiler_params=_PARAMS,
    scratch_types=[
        pltpu.VMEM((EW,), jnp.float32),
        pltpu.VMEM((EW,), jnp.int32),
        pltpu.VMEM((NF, BATCH), jnp.float32),
        pltpu.VMEM((NF,), jnp.int32),
        pltpu.VMEM_SHARED((NF, BATCH), jnp.float32),
    ],
)(_sc_norm_body)


ROWS_BLK = 1024


def _finish_body(acc_ref, nf_ref, out_ref):
    a = acc_ref[0] + acc_ref[1]
    wb = nf_ref[0] + nf_ref[1]
    out_ref[...] = jnp.log(a / wb)


_finish_call = pl.pallas_call(
    _finish_body,
    grid=(N_PAD // ROWS_BLK,),
    in_specs=[
        pl.BlockSpec((NC, ROWS_BLK, BATCH), lambda i: (0, i, 0)),
        pl.BlockSpec((NC, ROWS_BLK, BATCH), lambda i: (0, i, 0)),
    ],
    out_specs=pl.BlockSpec((ROWS_BLK, BATCH), lambda i: (i, 0)),
    out_shape=jax.ShapeDtypeStruct((N_PAD, BATCH), jnp.float32),
)


def kernel(data, log_weights, segment_ids):
    ids32 = segment_ids.astype(jnp.int32)
    ids3 = ids32.reshape(NW, CH, K)
    lw = log_weights.astype(jnp.float32)
    acc = _sc_main_call(data, lw, ids3)
    accw = _sc_norm_call(lw, ids32)
    acc = acc.reshape(NC, N_PAD, BATCH)
    accw = jnp.broadcast_to(
        accw.reshape(NC, N_PAD)[:, :, None], (NC, N_PAD, BATCH))
    return _finish_call(acc, accw)[:N_NODES]


# speedup vs baseline: 17.4313x; 1.0392x over previous
import functools

import jax
import jax.numpy as jnp
from jax import lax
from jax.experimental import pallas as pl
from jax.experimental.pallas import tpu as pltpu
from jax.experimental.pallas import tpu_sc as plsc

N_NODES = 10000
N_EDGES = 320000
BATCH = 128

NC, NS, LANES = 2, 16, 16
NW = NC * NS
EW = N_EDGES // NW
K = 80
CH = EW // K
N_PAD = 10240
RPS = N_PAD // NS
NF = N_PAD // BATCH

_MESH = plsc.VectorSubcoreMesh(core_axis_name="c", subcore_axis_name="s",
                               num_cores=NC, num_subcores=NS)
_PARAMS = pltpu.CompilerParams(needs_layout_passes=False)


def _sc_main_body(data_h, lw_h, ids3_h, acc_out_h,
                  data_v, idx_v, lw_c0, lw_c1, acc_sh,
                  sem0, sem1, sem2, sem3):
    c = lax.axis_index("c")
    s = lax.axis_index("s")
    w = c * NS + s
    base = w * EW

    def zrow(i, carry):
        for h in range(BATCH // LANES):
            data_v[i, pl.ds(h * LANES, LANES)] = jnp.zeros((LANES,), jnp.float32)
        return carry

    lax.fori_loop(0, 2 * K, zrow, 0, unroll=False)

    arow = s * RPS
    for i in range(4):
        pltpu.sync_copy(data_v, acc_sh.at[pl.ds(arow + i * 2 * K, 2 * K)])
    plsc.subcore_barrier()

    pltpu.sync_copy(ids3_h.at[w], idx_v)

    sems = ((sem0, sem1), (sem2, sem3))
    lwbufs = (lw_c0, lw_c1)
    H = K // 2

    def gather_descs(chunk, b):
        off = base + chunk * K
        return (
            pltpu.make_async_copy(data_h.at[pl.ds(off, H)],
                                  data_v.at[pl.ds(K * b, H)], sems[b][0]),
            pltpu.make_async_copy(data_h.at[pl.ds(off + H, H)],
                                  data_v.at[pl.ds(K * b + H, H)], sems[b][1]),
            pltpu.make_async_copy(lw_h.at[pl.ds(off, K)], lwbufs[b],
                                  sems[b][0]),
        )

    def gather_start(chunk, b):
        for d in gather_descs(chunk, b):
            d.start()

    def gather_wait(chunk, b):
        for d in gather_descs(chunk, b):
            d.wait()

    def compute(chunk, b):
        lwbuf = lwbufs[b]

        def ebody(k, carry):
            kv = jnp.broadcast_to(k, (LANES,)).astype(jnp.int32)
            lwb = plsc.load_gather(lwbuf, [kv])
            row = K * b + k
            for h in range(BATCH // LANES):
                x = data_v[row, pl.ds(h * LANES, LANES)]
                data_v[row, pl.ds(h * LANES, LANES)] = jnp.exp(x + lwb)
            return carry

        lax.fori_loop(0, K, ebody, 0, unroll=False)

    def compute_and_scatter(chunk, b):
        compute(chunk, b)
        pltpu.sync_copy(data_v.at[pl.ds(K * b, K)],
                        acc_sh.at[idx_v.at[chunk]], add=True)

    gather_start(0, 0)

    def gloop(g, carry):
        for b in range(2):
            chunk = 2 * g + b
            gather_wait(chunk, b)
            gather_start(chunk + 1, 1 - b)
            compute_and_scatter(chunk, b)
        return carry

    lax.fori_loop(0, (CH - 1) // 2, gloop, 0, unroll=False)
    gather_wait(CH - 1, 0)
    compute_and_scatter(CH - 1, 0)

    plsc.subcore_barrier()
    out_base = c * N_PAD + arow
    for i in range(4):
        pltpu.sync_copy(acc_sh.at[pl.ds(arow + i * 2 * K, 2 * K)], data_v)
        pltpu.sync_copy(data_v, acc_out_h.at[pl.ds(out_base + i * 2 * K, 2 * K)])


_sc_main_call = functools.partial(
    pl.kernel,
    out_type=jax.ShapeDtypeStruct((NC * N_PAD, BATCH), jnp.float32),
    mesh=_MESH,
    compiler_params=_PARAMS,
    scratch_types=[
        pltpu.VMEM((2 * K, BATCH), jnp.float32),
        pltpu.VMEM((CH, K), jnp.int32),
        pltpu.VMEM((K,), jnp.float32),
        pltpu.VMEM((K,), jnp.float32),
        pltpu.VMEM_SHARED((N_PAD, BATCH), jnp.float32),
        pltpu.SemaphoreType.DMA,
        pltpu.SemaphoreType.DMA,
        pltpu.SemaphoreType.DMA,
        pltpu.SemaphoreType.DMA,
    ],
)(_sc_main_body)


def _sc_norm_body(lw_h, ids_h, accw_out_h, lw_v, ids_v, norm2d, rowidx_v,
                  accw_sh):
    c = lax.axis_index("c")
    s = lax.axis_index("s")
    w = c * NS + s
    base = w * EW

    def zrow(i, carry):
        for h in range(BATCH // LANES):
            norm2d[i, pl.ds(h * LANES, LANES)] = jnp.zeros((LANES,), jnp.float32)
        return carry

    lax.fori_loop(0, NF, zrow, 0, unroll=False)

    @pl.when(s == 0)
    def _():
        pltpu.sync_copy(norm2d, accw_sh)

    for i in range(NF // LANES):
        rowidx_v[pl.ds(i * LANES, LANES)] = (
            lax.iota(jnp.int32, LANES) + jnp.int32(i * LANES))
    plsc.subcore_barrier()

    pltpu.sync_copy(lw_h.at[pl.ds(base, EW)], lw_v)
    pltpu.sync_copy(ids_h.at[pl.ds(base, EW)], ids_v)

    def ebody(g, carry):
        idv = ids_v[pl.ds(g * LANES, LANES)]
        v = jnp.exp(lw_v[pl.ds(g * LANES, LANES)])
        plsc.addupdate_scatter(
            norm2d, [lax.shift_right_logical(idv, 7), idv & 127], v)
        return carry

    lax.fori_loop(0, EW // LANES, ebody, 0, unroll=False)

    pltpu.sync_copy(norm2d, accw_sh.at[rowidx_v], add=True)
    plsc.subcore_barrier()

    @pl.when(s < NF // 8)
    def _():
        pltpu.sync_copy(accw_sh.at[pl.ds(s * 8, 8)],
                        accw_out_h.at[pl.ds(c * NF + s * 8, 8)])


_sc_norm_call = functools.partial(
    pl.kernel,
    out_type=jax.ShapeDtypeStruct((NC * NF, BATCH), jnp.float32),
    mesh=_MESH,
    compiler_params=_PARAMS,
    scratch_types=[
        pltpu.VMEM((EW,), jnp.float32),
        pltpu.VMEM((EW,), jnp.int32),
        pltpu.VMEM((NF, BATCH), jnp.float32),
        pltpu.VMEM((NF,), jnp.int32),
        pltpu.VMEM_SHARED((NF, BATCH), jnp.float32),
    ],
)(_sc_norm_body)


ROWS_BLK = 1024


def _finish_body(acc_ref, nf_ref, out_ref):
    a = acc_ref[0] + acc_ref[1]
    wb = nf_ref[0] + nf_ref[1]
    out_ref[...] = jnp.log(a / wb)


_finish_call = pl.pallas_call(
    _finish_body,
    grid=(N_PAD // ROWS_BLK,),
    in_specs=[
        pl.BlockSpec((NC, ROWS_BLK, BATCH), lambda i: (0, i, 0)),
        pl.BlockSpec((NC, ROWS_BLK, BATCH), lambda i: (0, i, 0)),
    ],
    out_specs=pl.BlockSpec((ROWS_BLK, BATCH), lambda i: (i, 0)),
    out_shape=jax.ShapeDtypeStruct((N_PAD, BATCH), jnp.float32),
)


def kernel(data, log_weights, segment_ids):
    ids32 = segment_ids.astype(jnp.int32)
    ids3 = ids32.reshape(NW, CH, K)
    lw = log_weights.astype(jnp.float32)
    acc = _sc_main_call(data, lw, ids3)
    accw = _sc_norm_call(lw, ids32)
    acc = acc.reshape(NC, N_PAD, BATCH)
    accw = jnp.broadcast_to(
        accw.reshape(NC, N_PAD)[:, :, None], (NC, N_PAD, BATCH))
    return _finish_call(acc, accw)[:N_NODES]


# speedup vs baseline: 17.4550x; 1.0014x over previous
import functools

import jax
import jax.numpy as jnp
from jax import lax
from jax.experimental import pallas as pl
from jax.experimental.pallas import tpu as pltpu
from jax.experimental.pallas import tpu_sc as plsc

N_NODES = 10000
N_EDGES = 320000
BATCH = 128

NC, NS, LANES = 2, 16, 16
NW = NC * NS
EW = N_EDGES // NW
K = 80
CH = EW // K
N_PAD = 10240
RPS = N_PAD // NS
NF = N_PAD // BATCH

_MESH = plsc.VectorSubcoreMesh(core_axis_name="c", subcore_axis_name="s",
                               num_cores=NC, num_subcores=NS)
_PARAMS = pltpu.CompilerParams(needs_layout_passes=False)


def _sc_main_body(data_h, lw_h, ids3_h, acc_out_h,
                  data_v, idx_v, lw_c0, lw_c1, acc_sh, sem0, sem1):
    c = lax.axis_index("c")
    s = lax.axis_index("s")
    w = c * NS + s
    base = w * EW

    def zrow(i, carry):
        for h in range(BATCH // LANES):
            data_v[i, pl.ds(h * LANES, LANES)] = jnp.zeros((LANES,), jnp.float32)
        return carry

    lax.fori_loop(0, 2 * K, zrow, 0, unroll=False)

    arow = s * RPS
    for i in range(4):
        pltpu.sync_copy(data_v, acc_sh.at[pl.ds(arow + i * 2 * K, 2 * K)])
    plsc.subcore_barrier()

    pltpu.sync_copy(ids3_h.at[w], idx_v)

    sems = (sem0, sem1)
    lwbufs = (lw_c0, lw_c1)

    def gather_descs(chunk, b):
        off = base + chunk * K
        return (
            pltpu.make_async_copy(data_h.at[w * CH + chunk],
                                  data_v.at[pl.ds(K * b, K)], sems[b]),
            pltpu.make_async_copy(lw_h.at[pl.ds(off, K)], lwbufs[b], sems[b]),
        )

    def gather_start(chunk, b):
        for d in gather_descs(chunk, b):
            d.start()

    def gather_wait(chunk, b):
        for d in gather_descs(chunk, b):
            d.wait()

    def compute(chunk, b):
        lwbuf = lwbufs[b]

        def ebody(k, carry):
            kv = jnp.broadcast_to(k, (LANES,)).astype(jnp.int32)
            lwb = plsc.load_gather(lwbuf, [kv])
            row = K * b + k
            for h in range(BATCH // LANES):
                x = data_v[row, pl.ds(h * LANES, LANES)]
                data_v[row, pl.ds(h * LANES, LANES)] = jnp.exp(x + lwb)
            return carry

        lax.fori_loop(0, K, ebody, 0, unroll=False)

    def compute_and_scatter(chunk, b):
        compute(chunk, b)
        pltpu.sync_copy(data_v.at[pl.ds(K * b, K)],
                        acc_sh.at[idx_v.at[chunk]], add=True)

    gather_start(0, 0)

    def gloop(g, carry):
        for b in range(2):
            chunk = 2 * g + b
            gather_wait(chunk, b)
            gather_start(chunk + 1, 1 - b)
            compute_and_scatter(chunk, b)
        return carry

    lax.fori_loop(0, (CH - 1) // 2, gloop, 0, unroll=False)
    gather_wait(CH - 1, 0)
    compute_and_scatter(CH - 1, 0)

    plsc.subcore_barrier()
    out_base = c * N_PAD + arow
    for i in range(4):
        pltpu.sync_copy(acc_sh.at[pl.ds(arow + i * 2 * K, 2 * K)], data_v)
        pltpu.sync_copy(data_v, acc_out_h.at[pl.ds(out_base + i * 2 * K, 2 * K)])


_sc_main_call = functools.partial(
    pl.kernel,
    out_type=jax.ShapeDtypeStruct((NC * N_PAD, BATCH), jnp.float32),
    mesh=_MESH,
    compiler_params=_PARAMS,
    scratch_types=[
        pltpu.VMEM((2 * K, BATCH), jnp.float32),
        pltpu.VMEM((CH, K), jnp.int32),
        pltpu.VMEM((K,), jnp.float32),
        pltpu.VMEM((K,), jnp.float32),
        pltpu.VMEM_SHARED((N_PAD, BATCH), jnp.float32),
        pltpu.SemaphoreType.DMA,
        pltpu.SemaphoreType.DMA,
    ],
)(_sc_main_body)


def _sc_norm_body(lw_h, ids_h, accw_out_h, lw_v, ids_v, norm2d, rowidx_v,
                  accw_sh):
    c = lax.axis_index("c")
    s = lax.axis_index("s")
    w = c * NS + s
    base = w * EW

    def zrow(i, carry):
        for h in range(BATCH // LANES):
            norm2d[i, pl.ds(h * LANES, LANES)] = jnp.zeros((LANES,), jnp.float32)
        return carry

    lax.fori_loop(0, NF, zrow, 0, unroll=False)

    @pl.when(s == 0)
    def _():
        pltpu.sync_copy(norm2d, accw_sh)

    for i in range(NF // LANES):
        rowidx_v[pl.ds(i * LANES, LANES)] = (
            lax.iota(jnp.int32, LANES) + jnp.int32(i * LANES))
    plsc.subcore_barrier()

    pltpu.sync_copy(lw_h.at[pl.ds(base, EW)], lw_v)
    pltpu.sync_copy(ids_h.at[pl.ds(base, EW)], ids_v)

    def ebody(g, carry):
        idv = ids_v[pl.ds(g * LANES, LANES)]
        v = jnp.exp(lw_v[pl.ds(g * LANES, LANES)])
        plsc.addupdate_scatter(
            norm2d, [lax.shift_right_logical(idv, 7), idv & 127], v)
        return carry

    lax.fori_loop(0, EW // LANES, ebody, 0, unroll=False)

    pltpu.sync_copy(norm2d, accw_sh.at[rowidx_v], add=True)
    plsc.subcore_barrier()

    @pl.when(s < NF // 8)
    def _():
        pltpu.sync_copy(accw_sh.at[pl.ds(s * 8, 8)],
                        accw_out_h.at[pl.ds(c * NF + s * 8, 8)])


_sc_norm_call = functools.partial(
    pl.kernel,
    out_type=jax.ShapeDtypeStruct((NC * NF, BATCH), jnp.float32),
    mesh=_MESH,
    compiler_params=_PARAMS,
    scratch_types=[
        pltpu.VMEM((EW,), jnp.float32),
        pltpu.VMEM((EW,), jnp.int32),
        pltpu.VMEM((NF, BATCH), jnp.float32),
        pltpu.VMEM((NF,), jnp.int32),
        pltpu.VMEM_SHARED((NF, BATCH), jnp.float32),
    ],
)(_sc_norm_body)


ROWS_BLK = 1024


def _finish_body(acc_ref, nf_ref, out_ref):
    a = acc_ref[0] + acc_ref[1]
    wb = nf_ref[0] + nf_ref[1]
    out_ref[...] = jnp.log(a / wb)


_finish_call = pl.pallas_call(
    _finish_body,
    grid=(N_PAD // ROWS_BLK,),
    in_specs=[
        pl.BlockSpec((NC, ROWS_BLK, BATCH), lambda i: (0, i, 0)),
        pl.BlockSpec((NC, ROWS_BLK, BATCH), lambda i: (0, i, 0)),
    ],
    out_specs=pl.BlockSpec((ROWS_BLK, BATCH), lambda i: (i, 0)),
    out_shape=jax.ShapeDtypeStruct((N_PAD, BATCH), jnp.float32),
)


def kernel(data, log_weights, segment_ids):
    ids32 = segment_ids.astype(jnp.int32)
    ids3 = ids32.reshape(NW, CH, K)
    lw = log_weights.astype(jnp.float32)
    acc = _sc_main_call(data.reshape(N_EDGES // K, K, BATCH), lw, ids3)
    accw = _sc_norm_call(lw, ids32)
    acc = acc.reshape(NC, N_PAD, BATCH)
    accw = jnp.broadcast_to(
        accw.reshape(NC, N_PAD)[:, :, None], (NC, N_PAD, BATCH))
    return _finish_call(acc, accw)[:N_NODES]


# speedup vs baseline: 17.4585x; 1.0002x over previous
import functools

import jax
import jax.numpy as jnp
from jax import lax
from jax.experimental import pallas as pl
from jax.experimental.pallas import tpu as pltpu
from jax.experimental.pallas import tpu_sc as plsc

N_NODES = 10000
N_EDGES = 320000
BATCH = 128

NC, NS, LANES = 2, 16, 16
NW = NC * NS
EW = N_EDGES // NW
K = 80
CH = EW // K
N_PAD = 10240
RPS = N_PAD // NS
NF = N_PAD // BATCH

_MESH = plsc.VectorSubcoreMesh(core_axis_name="c", subcore_axis_name="s",
                               num_cores=NC, num_subcores=NS)
_PARAMS = pltpu.CompilerParams(needs_layout_passes=False)


def _sc_main_body(data_h, lw_h, ids3_h, acc_out_h,
                  data_v, idx_v, lw_c0, lw_c1, acc_sh, sem0, sem1):
    c = lax.axis_index("c")
    s = lax.axis_index("s")
    w = c * NS + s
    base = w * EW

    def zrow(i, carry):
        for h in range(BATCH // LANES):
            data_v[i, pl.ds(h * LANES, LANES)] = jnp.zeros((LANES,), jnp.float32)
        return carry

    lax.fori_loop(0, 2 * K, zrow, 0, unroll=False)

    arow = s * RPS
    for i in range(4):
        pltpu.sync_copy(data_v, acc_sh.at[pl.ds(arow + i * 2 * K, 2 * K)])
    plsc.subcore_barrier()

    pltpu.sync_copy(ids3_h.at[w], idx_v)

    sems = (sem0, sem1)
    lwbufs = (lw_c0, lw_c1)

    def gather_descs(chunk, b):
        off = base + chunk * K
        return (
            pltpu.make_async_copy(data_h.at[pl.ds(off, K)],
                                  data_v.at[pl.ds(K * b, K)], sems[b]),
            pltpu.make_async_copy(lw_h.at[pl.ds(off, K)], lwbufs[b], sems[b]),
        )

    def gather_start(chunk, b):
        for d in gather_descs(chunk, b):
            d.start()

    def gather_wait(chunk, b):
        for d in gather_descs(chunk, b):
            d.wait()

    def compute(chunk, b):
        lwbuf = lwbufs[b]

        def ebody(k, carry):
            kv = jnp.broadcast_to(k, (LANES,)).astype(jnp.int32)
            lwb = plsc.load_gather(lwbuf, [kv])
            row = K * b + k
            for h in range(BATCH // LANES):
                x = data_v[row, pl.ds(h * LANES, LANES)]
                data_v[row, pl.ds(h * LANES, LANES)] = jnp.exp(x + lwb)
            return carry

        lax.fori_loop(0, K, ebody, 0, unroll=False)

    def compute_and_scatter(chunk, b):
        compute(chunk, b)
        pltpu.sync_copy(data_v.at[pl.ds(K * b, K)],
                        acc_sh.at[idx_v.at[chunk]], add=True)

    gather_start(0, 0)

    def gloop(g, carry):
        for b in range(2):
            chunk = 2 * g + b
            gather_wait(chunk, b)
            gather_start(chunk + 1, 1 - b)
            compute_and_scatter(chunk, b)
        return carry

    lax.fori_loop(0, (CH - 1) // 2, gloop, 0, unroll=False)
    gather_wait(CH - 1, 0)
    compute_and_scatter(CH - 1, 0)

    plsc.subcore_barrier()
    out_base = c * N_PAD + arow
    for i in range(4):
        pltpu.sync_copy(acc_sh.at[pl.ds(arow + i * 2 * K, 2 * K)], data_v)
        pltpu.sync_copy(data_v, acc_out_h.at[pl.ds(out_base + i * 2 * K, 2 * K)])


_sc_main_call = functools.partial(
    pl.kernel,
    out_type=jax.ShapeDtypeStruct((NC * N_PAD, BATCH), jnp.float32),
    mesh=_MESH,
    compiler_params=_PARAMS,
    scratch_types=[
        pltpu.VMEM((2 * K, BATCH), jnp.float32),
        pltpu.VMEM((CH, K), jnp.int32),
        pltpu.VMEM((K,), jnp.float32),
        pltpu.VMEM((K,), jnp.float32),
        pltpu.VMEM_SHARED((N_PAD, BATCH), jnp.float32),
        pltpu.SemaphoreType.DMA,
        pltpu.SemaphoreType.DMA,
    ],
)(_sc_main_body)


def _sc_norm_body(lw_h, ids_h, accw_out_h, lw_v, ids_v, norm2d, rowidx_v,
                  accw_sh):
    c = lax.axis_index("c")
    s = lax.axis_index("s")
    w = c * NS + s
    base = w * EW

    def zrow(i, carry):
        for h in range(BATCH // LANES):
            norm2d[i, pl.ds(h * LANES, LANES)] = jnp.zeros((LANES,), jnp.float32)
        return carry

    lax.fori_loop(0, NF, zrow, 0, unroll=False)

    @pl.when(s == 0)
    def _():
        pltpu.sync_copy(norm2d, accw_sh)

    for i in range(NF // LANES):
        rowidx_v[pl.ds(i * LANES, LANES)] = (
            lax.iota(jnp.int32, LANES) + jnp.int32(i * LANES))
    plsc.subcore_barrier()

    pltpu.sync_copy(lw_h.at[pl.ds(base, EW)], lw_v)
    pltpu.sync_copy(ids_h.at[pl.ds(base, EW)], ids_v)

    def ebody(g, carry):
        idv = ids_v[pl.ds(g * LANES, LANES)]
        v = jnp.exp(lw_v[pl.ds(g * LANES, LANES)])
        plsc.addupdate_scatter(
            norm2d, [lax.shift_right_logical(idv, 7), idv & 127], v)
        return carry

    lax.fori_loop(0, EW // LANES, ebody, 0, unroll=False)

    pltpu.sync_copy(norm2d, accw_sh.at[rowidx_v], add=True)
    plsc.subcore_barrier()

    @pl.when(s < NF // 8)
    def _():
        pltpu.sync_copy(accw_sh.at[pl.ds(s * 8, 8)],
                        accw_out_h.at[pl.ds(c * NF + s * 8, 8)])


_sc_norm_call = functools.partial(
    pl.kernel,
    out_type=jax.ShapeDtypeStruct((NC * NF, BATCH), jnp.float32),
    mesh=_MESH,
    compiler_params=_PARAMS,
    scratch_types=[
        pltpu.VMEM((EW,), jnp.float32),
        pltpu.VMEM((EW,), jnp.int32),
        pltpu.VMEM((NF, BATCH), jnp.float32),
        pltpu.VMEM((NF,), jnp.int32),
        pltpu.VMEM_SHARED((NF, BATCH), jnp.float32),
    ],
)(_sc_norm_body)


ROWS_BLK = 1024


def _finish_body(acc_ref, nf_ref, out_ref):
    a = acc_ref[0] + acc_ref[1]
    wb = nf_ref[0] + nf_ref[1]
    out_ref[...] = jnp.log(a / wb)


_finish_call = pl.pallas_call(
    _finish_body,
    grid=(N_PAD // ROWS_BLK,),
    in_specs=[
        pl.BlockSpec((NC, ROWS_BLK, BATCH), lambda i: (0, i, 0)),
        pl.BlockSpec((NC, ROWS_BLK, BATCH), lambda i: (0, i, 0)),
    ],
    out_specs=pl.BlockSpec((ROWS_BLK, BATCH), lambda i: (i, 0)),
    out_shape=jax.ShapeDtypeStruct((N_PAD, BATCH), jnp.float32),
)


def kernel(data, log_weights, segment_ids):
    ids32 = segment_ids.astype(jnp.int32)
    ids3 = ids32.reshape(NW, CH, K)
    lw = log_weights.astype(jnp.float32)
    acc = _sc_main_call(data, lw, ids3)
    accw = _sc_norm_call(lw, ids32)
    acc = acc.reshape(NC, N_PAD, BATCH)
    accw = jnp.broadcast_to(
        accw.reshape(NC, N_PAD)[:, :, None], (NC, N_PAD, BATCH))
    return _finish_call(acc, accw)[:N_NODES]
